# baseline (speedup 1.0000x reference)
import jax
import jax.numpy as jnp
from jax.experimental import pallas as pl
from jax.experimental.pallas import tpu as pltpu

_CLS_PAD = 128
_VMEM_LIMIT = 44 * 1024 * 1024
_H1 = 1440
_H2 = 320
_K1 = 28 * 28


def _fused_body(x_ref, a_ref, b1_ref, c_ref, b2_ref, wfc_ref, bfc_ref, o_ref):
    xb = x_ref[...].astype(jnp.bfloat16)

    z = jnp.dot(xb, a_ref[...], preferred_element_type=jnp.float32)
    m = jnp.maximum(jnp.maximum(z[:, :_H1], z[:, _H1:2 * _H1]),
                    jnp.maximum(z[:, 2 * _H1:3 * _H1], z[:, 3 * _H1:]))
    h1 = jnp.maximum(m + b1_ref[...], 0.0).astype(jnp.bfloat16)

    z2 = jnp.dot(h1, c_ref[...], preferred_element_type=jnp.float32)
    m2 = jnp.maximum(jnp.maximum(z2[:, :_H2], z2[:, _H2:2 * _H2]),
                     jnp.maximum(z2[:, 2 * _H2:3 * _H2], z2[:, 3 * _H2:]))
    h2 = jnp.maximum(m2 + b2_ref[...], 0.0).astype(jnp.bfloat16)

    logits = jnp.dot(h2, wfc_ref[...],
                     preferred_element_type=jnp.float32) + bfc_ref[...]
    mx = jnp.max(logits, axis=-1, keepdims=True)
    s = logits - mx
    lse = jnp.log(jnp.sum(jnp.exp(s), axis=-1, keepdims=True))
    o_ref[...] = s - lse


def kernel(x, a00, a01, a10, a11, b1, c00, c01, c10, c11, b2, wfc, bfc):
    n = x.shape[0]
    x2d = x.reshape(n, _K1)
    a_cat = jnp.concatenate([a00, a01, a10, a11], axis=1)
    c_cat = jnp.concatenate([c00, c01, c10, c11], axis=1)

    tb = 256 if n >= 256 else 8 * pl.cdiv(n, 8)
    grid = pl.cdiv(n, tb)
    n_pad = grid * tb
    if n_pad != n:
        x2d = jnp.pad(x2d, ((0, n_pad - n), (0, 0)))

    def const_spec(arr):
        return pl.BlockSpec(arr.shape, lambda i: (0, 0))

    weights = [a_cat, b1, c_cat, b2, wfc, bfc]
    out = pl.pallas_call(
        _fused_body,
        out_shape=jax.ShapeDtypeStruct((n_pad, _CLS_PAD), jnp.float32),
        grid=(grid,),
        in_specs=[pl.BlockSpec((tb, _K1), lambda i: (i, 0))]
                 + [const_spec(w) for w in weights],
        out_specs=pl.BlockSpec((tb, _CLS_PAD), lambda i: (i, 0)),
        compiler_params=pltpu.CompilerParams(
            dimension_semantics=("parallel",),
            vmem_limit_bytes=_VMEM_LIMIT),
    )(x2d, *weights)
    return out[:n, :10]


# speedup vs baseline: 1.0224x; 1.0224x over previous
import jax
import jax.numpy as jnp
from jax.experimental import pallas as pl
from jax.experimental.pallas import tpu as pltpu

_CLS_PAD = 128
_VMEM_LIMIT = 44 * 1024 * 1024
_H1 = 1440
_H2 = 320
_K1 = 28 * 28


def _fused_body(x_ref, a_ref, b1_ref, c_ref, b2_ref, wfc_ref, bfc_ref, o_ref):
    xb = x_ref[...]

    z = jnp.dot(xb, a_ref[...], preferred_element_type=jnp.float32)
    m = jnp.maximum(jnp.maximum(z[:, :_H1], z[:, _H1:2 * _H1]),
                    jnp.maximum(z[:, 2 * _H1:3 * _H1], z[:, 3 * _H1:]))
    h1 = jnp.maximum(m + b1_ref[...], 0.0).astype(jnp.bfloat16)

    z2 = jnp.dot(h1, c_ref[...], preferred_element_type=jnp.float32)
    m2 = jnp.maximum(jnp.maximum(z2[:, :_H2], z2[:, _H2:2 * _H2]),
                     jnp.maximum(z2[:, 2 * _H2:3 * _H2], z2[:, 3 * _H2:]))
    h2 = jnp.maximum(m2 + b2_ref[...], 0.0).astype(jnp.bfloat16)

    logits = jnp.dot(h2, wfc_ref[...],
                     preferred_element_type=jnp.float32) + bfc_ref[...]
    mx = jnp.max(logits, axis=-1, keepdims=True)
    s = logits - mx
    lse = jnp.log(jnp.sum(jnp.exp(s), axis=-1, keepdims=True))
    o_ref[...] = (s - lse)[:, :10]


def kernel(x, a00, a01, a10, a11, b1, c00, c01, c10, c11, b2, wfc, bfc):
    n = x.shape[0]
    x2d = x.reshape(n, _K1).astype(jnp.bfloat16)
    a_cat = jnp.concatenate([a00, a01, a10, a11], axis=1)
    c_cat = jnp.concatenate([c00, c01, c10, c11], axis=1)

    tb = 256 if n >= 256 else 8 * pl.cdiv(n, 8)
    grid = pl.cdiv(n, tb)
    n_pad = grid * tb
    if n_pad != n:
        x2d = jnp.pad(x2d, ((0, n_pad - n), (0, 0)))

    def const_spec(arr):
        return pl.BlockSpec(arr.shape, lambda i: (0, 0))

    weights = [a_cat, b1, c_cat, b2, wfc, bfc]
    out = pl.pallas_call(
        _fused_body,
        out_shape=jax.ShapeDtypeStruct((n_pad, 10), jnp.float32),
        grid=(grid,),
        in_specs=[pl.BlockSpec((tb, _K1), lambda i: (i, 0))]
                 + [const_spec(w) for w in weights],
        out_specs=pl.BlockSpec((tb, 10), lambda i: (i, 0)),
        compiler_params=pltpu.CompilerParams(
            dimension_semantics=("parallel",),
            vmem_limit_bytes=_VMEM_LIMIT),
    )(x2d, *weights)
    return out[:n]


# speedup vs baseline: 1.1012x; 1.0770x over previous
import jax
import jax.numpy as jnp
from jax.experimental import pallas as pl
from jax.experimental.pallas import tpu as pltpu

_CLS_PAD = 128
_VMEM_LIMIT = 44 * 1024 * 1024
_H1 = 1440
_H2 = 320
_K1 = 28 * 28


def _fused_body(x_ref, a_ref, b1_ref, c_ref, b2_ref, wfc_ref, bfc_ref, o_ref):
    xb = x_ref[...]

    z = jnp.dot(xb, a_ref[...], preferred_element_type=jnp.float32)
    m = jnp.maximum(jnp.maximum(z[:, :_H1], z[:, _H1:2 * _H1]),
                    jnp.maximum(z[:, 2 * _H1:3 * _H1], z[:, 3 * _H1:]))
    h1 = jnp.maximum(m + b1_ref[...], 0.0).astype(jnp.bfloat16)

    z2 = jnp.dot(h1, c_ref[...], preferred_element_type=jnp.float32)
    m2 = jnp.maximum(jnp.maximum(z2[:, :_H2], z2[:, _H2:2 * _H2]),
                     jnp.maximum(z2[:, 2 * _H2:3 * _H2], z2[:, 3 * _H2:]))
    h2 = jnp.maximum(m2 + b2_ref[...], 0.0).astype(jnp.bfloat16)

    logits = jnp.dot(h2, wfc_ref[...],
                     preferred_element_type=jnp.float32) + bfc_ref[...]
    mx = jnp.max(logits, axis=-1, keepdims=True)
    s = logits - mx
    lse = jnp.log(jnp.sum(jnp.exp(s), axis=-1, keepdims=True))
    o_ref[...] = (s - lse)[:, :10]


def kernel(x, a00, a01, a10, a11, b1, c00, c01, c10, c11, b2, wfc, bfc):
    n = x.shape[0]
    x2d = x.reshape(n, _K1).astype(jnp.bfloat16)
    a_cat = jnp.concatenate([a00, a01, a10, a11], axis=1)
    c_cat = jnp.concatenate([c00, c01, c10, c11], axis=1)

    tb = 512 if n >= 512 else 8 * pl.cdiv(n, 8)
    grid = pl.cdiv(n, tb)
    n_pad = grid * tb
    if n_pad != n:
        x2d = jnp.pad(x2d, ((0, n_pad - n), (0, 0)))

    def const_spec(arr):
        return pl.BlockSpec(arr.shape, lambda i: (0, 0))

    weights = [a_cat, b1, c_cat, b2, wfc, bfc]
    out = pl.pallas_call(
        _fused_body,
        out_shape=jax.ShapeDtypeStruct((n_pad, 10), jnp.float32),
        grid=(grid,),
        in_specs=[pl.BlockSpec((tb, _K1), lambda i: (i, 0))]
                 + [const_spec(w) for w in weights],
        out_specs=pl.BlockSpec((tb, 10), lambda i: (i, 0)),
        compiler_params=pltpu.CompilerParams(
            dimension_semantics=("parallel",),
            vmem_limit_bytes=_VMEM_LIMIT),
    )(x2d, *weights)
    return out[:n]
